# baseline (speedup 1.0000x reference)
import functools

import jax
import jax.numpy as jnp
from jax import lax
from jax.experimental import pallas as pl
from jax.experimental.pallas import tpu as pltpu
from jax.experimental.pallas import tpu_sc as plsc

B, S, C = 4, 4096, 1024
BS = 256
CH2 = 512


def _add_kernel(img_ref, aud_ref, pi_ref, pa_ref, oi_ref, oa_ref, pea_ref):
    pi = pi_ref[...]
    pa = pa_ref[...]
    pe_a = jnp.broadcast_to(pa[None], (B, BS, C))
    oi_ref[...] = img_ref[...] + jnp.broadcast_to(pi[None], (B, BS, C))
    oa_ref[...] = aud_ref[...] + pe_a
    pea_ref[...] = pe_a


def _tc_part(image, audio, pos_image, pos_audio):
    grid = (S // BS,)
    in_spec3 = pl.BlockSpec((B, BS, C), lambda s: (0, s, 0))
    in_spec2 = pl.BlockSpec((BS, C), lambda s: (s, 0))
    out_spec = pl.BlockSpec((B, BS, C), lambda s: (0, s, 0))
    out_shape = jax.ShapeDtypeStruct((B, S, C), jnp.float32)
    return pl.pallas_call(
        _add_kernel,
        grid=grid,
        in_specs=[in_spec3, in_spec3, in_spec2, in_spec2],
        out_specs=[out_spec, out_spec, out_spec],
        out_shape=[out_shape, out_shape, out_shape],
    )(image, audio, pos_image, pos_audio)


@functools.partial(
    pl.kernel,
    mesh=plsc.ScalarSubcoreMesh(axis_name="c"),
    out_type=jax.ShapeDtypeStruct((B, S, C), jnp.float32),
    scratch_types=[pltpu.VMEM_SHARED((CH2, C), jnp.float32),
                   pltpu.VMEM_SHARED((CH2, C), jnp.float32),
                   pltpu.SemaphoreType.DMA,
                   pltpu.SemaphoreType.DMA],
)
def _sc_broadcast(pos_hbm, pe_hbm, buf0, buf1, rsem, wsem):
    cid = lax.axis_index("c")
    half = S // 2
    base = cid * half
    bufs = (buf0, buf1)
    n = half // CH2
    pltpu.async_copy(pos_hbm.at[pl.ds(base, CH2), :], bufs[0], rsem)
    for i in range(n):
        r = base + i * CH2
        buf = bufs[i % 2]
        pltpu.make_async_copy(pos_hbm.at[pl.ds(r, CH2), :], buf, rsem).wait()
        if i + 1 < n:
            nbuf = bufs[(i + 1) % 2]
            if i >= 1:
                pr = base + (i - 1) * CH2
                for _ in range(B):
                    pltpu.make_async_copy(nbuf, pe_hbm.at[0, pl.ds(pr, CH2), :],
                                          wsem).wait()
            pltpu.async_copy(pos_hbm.at[pl.ds(r + CH2, CH2), :], nbuf, rsem)
        for b in range(B):
            pltpu.async_copy(buf, pe_hbm.at[b, pl.ds(r, CH2), :], wsem)
    for i in (n - 2, n - 1):
        r = base + i * CH2
        for _ in range(B):
            pltpu.make_async_copy(bufs[i % 2], pe_hbm.at[0, pl.ds(r, CH2), :],
                                  wsem).wait()


def kernel(image, audio, pos_image, pos_audio):
    out_image, out_audio, pe_audio = _tc_part(image, audio, pos_image,
                                              pos_audio)
    pe_image = _sc_broadcast(pos_image)
    return (out_image, out_audio, pe_image, pe_audio)


# speedup vs baseline: 1.1550x; 1.1550x over previous
import jax
import jax.numpy as jnp
from jax.experimental import pallas as pl

B, S, C = 4, 4096, 1024
BS = 256


def _pe_kernel(img_ref, aud_ref, pi_ref, pa_ref,
               oi_ref, oa_ref, pei_ref, pea_ref):
    pi = pi_ref[...]
    pa = pa_ref[...]
    pe_i = jnp.broadcast_to(pi[None], (B, BS, C))
    pe_a = jnp.broadcast_to(pa[None], (B, BS, C))
    oi_ref[...] = img_ref[...] + pe_i
    oa_ref[...] = aud_ref[...] + pe_a
    pei_ref[...] = pe_i
    pea_ref[...] = pe_a


def kernel(image, audio, pos_image, pos_audio):
    grid = (S // BS,)
    in_spec3 = pl.BlockSpec((B, BS, C), lambda s: (0, s, 0))
    in_spec2 = pl.BlockSpec((BS, C), lambda s: (s, 0))
    out_spec = pl.BlockSpec((B, BS, C), lambda s: (0, s, 0))
    out_shape = jax.ShapeDtypeStruct((B, S, C), jnp.float32)
    return pl.pallas_call(
        _pe_kernel,
        grid=grid,
        in_specs=[in_spec3, in_spec3, in_spec2, in_spec2],
        out_specs=[out_spec, out_spec, out_spec, out_spec],
        out_shape=[out_shape, out_shape, out_shape, out_shape],
    )(image, audio, pos_image, pos_audio)
